# baseline (speedup 1.0000x reference)
import functools

import jax
import jax.numpy as jnp
from jax.experimental import pallas as pl
from jax.experimental.pallas import tpu as pltpu


def _make_body(n, d_out, m_blk, depth, o_slots):
    n_blk = n // m_blk

    def _body(adj_hbm, x_hbm, w_ref, out_hbm,
              xbuf, buf, obuf, sems, osems, xsem):
        def cp_adj(i, slot):
            return pltpu.make_async_copy(
                adj_hbm.at[pl.ds(i * m_blk, m_blk), :],
                buf.at[slot],
                sems.at[slot],
            )

        def cp_out(i, oslot):
            return pltpu.make_async_copy(
                obuf.at[oslot],
                out_hbm.at[pl.ds(i * m_blk, m_blk), :],
                osems.at[oslot],
            )

        cp_x = pltpu.make_async_copy(x_hbm, xbuf, xsem)
        cp_x.start()

        for s in range(depth - 1):
            cp_adj(s, s).start()

        def step(i, carry):
            nxt = i + depth - 1

            @pl.when(nxt < n_blk)
            def _start_next():
                cp_adj(nxt, jax.lax.rem(nxt, depth)).start()

            slot = jax.lax.rem(i, depth)
            cp_adj(i, slot).wait()

            @pl.when(i == 0)
            def _wait_x():
                cp_x.wait()

            h = jnp.dot(buf[slot], xbuf[...],
                        preferred_element_type=jnp.float32)

            oslot = jax.lax.rem(i, o_slots)

            @pl.when(i >= o_slots)
            def _free_oslot():
                cp_out(i - o_slots, oslot).wait()

            obuf[oslot] = jax.lax.dot_general(
                h, w_ref[...], (((1,), (1,)), ((), ())),
                preferred_element_type=jnp.float32,
            )
            cp_out(i, oslot).start()
            return carry

        jax.lax.fori_loop(0, n_blk, step, 0)

        for tail in range(max(0, n_blk - o_slots), n_blk):
            cp_out(tail, tail % o_slots).wait()

    return _body


@functools.partial(jax.jit,
                   static_argnames=("m_blk", "depth", "o_slots", "interpret"))
def _graph_conv(x, adj, W, *, m_blk, depth, o_slots, interpret=False):
    n, d_in = x.shape
    d_out = W.shape[0]
    return pl.pallas_call(
        _make_body(n, d_out, m_blk, depth, o_slots),
        in_specs=[
            pl.BlockSpec(memory_space=pltpu.MemorySpace.HBM),
            pl.BlockSpec(memory_space=pltpu.MemorySpace.HBM),
            pl.BlockSpec(memory_space=pltpu.MemorySpace.VMEM),
        ],
        out_specs=pl.BlockSpec(memory_space=pltpu.MemorySpace.HBM),
        out_shape=jax.ShapeDtypeStruct((n, d_out), jnp.float32),
        scratch_shapes=[
            pltpu.VMEM((n, d_in), jnp.float32),
            pltpu.VMEM((depth, m_blk, n), jnp.float32),
            pltpu.VMEM((o_slots, m_blk, d_out), jnp.float32),
            pltpu.SemaphoreType.DMA((depth,)),
            pltpu.SemaphoreType.DMA((o_slots,)),
            pltpu.SemaphoreType.DMA,
        ],
        compiler_params=pltpu.CompilerParams(
            vmem_limit_bytes=64 * 1024 * 1024),
        interpret=interpret,
    )(adj, x, W)


def kernel(x, adj, W):
    n = x.shape[0]
    if n % 40 == 0 and n // 40 >= 16:
        return _graph_conv(x, adj, W, m_blk=40, depth=10, o_slots=8)
    return _graph_conv(x, adj, W, m_blk=n, depth=1, o_slots=1)


# speedup vs baseline: 1.5573x; 1.5573x over previous
import functools

import jax
import jax.numpy as jnp
from jax.experimental import pallas as pl
from jax.experimental.pallas import tpu as pltpu


def _make_body(n, d_out, m_blk, depth, o_slots):
    n_blk = n // m_blk

    def _body(adj_hbm, x_hbm, w_ref, out_hbm,
              xbuf, buf, obuf, sems, osems, xsem):
        def cp_adj(i, slot):
            return pltpu.make_async_copy(
                adj_hbm.at[pl.ds(i * m_blk, m_blk), :],
                buf.at[slot],
                sems.at[slot],
            )

        def cp_out(i, oslot):
            return pltpu.make_async_copy(
                obuf.at[oslot],
                out_hbm.at[pl.ds(i * m_blk, m_blk), :],
                osems.at[oslot],
            )

        cp_x = pltpu.make_async_copy(x_hbm, xbuf, xsem)
        cp_x.start()

        for s in range(depth - 1):
            cp_adj(s, s).start()

        def step(i, carry):
            nxt = i + depth - 1

            @pl.when(nxt < n_blk)
            def _start_next():
                cp_adj(nxt, jax.lax.rem(nxt, depth)).start()

            slot = jax.lax.rem(i, depth)
            cp_adj(i, slot).wait()

            @pl.when(i == 0)
            def _wait_x():
                cp_x.wait()

            h = jnp.dot(buf[slot], xbuf[...],
                        preferred_element_type=jnp.float32)

            oslot = jax.lax.rem(i, o_slots)

            @pl.when(i >= o_slots)
            def _free_oslot():
                cp_out(i - o_slots, oslot).wait()

            obuf[oslot] = jax.lax.dot_general(
                h, w_ref[...], (((1,), (1,)), ((), ())),
                preferred_element_type=jnp.float32,
            )
            cp_out(i, oslot).start()
            return carry

        jax.lax.fori_loop(0, n_blk, step, 0)

        for tail in range(max(0, n_blk - o_slots), n_blk):
            cp_out(tail, tail % o_slots).wait()

    return _body


@functools.partial(jax.jit,
                   static_argnames=("m_blk", "depth", "o_slots", "interpret"))
def _graph_conv(x, adj, W, *, m_blk, depth, o_slots, interpret=False):
    n, d_in = x.shape
    d_out = W.shape[0]
    return pl.pallas_call(
        _make_body(n, d_out, m_blk, depth, o_slots),
        in_specs=[
            pl.BlockSpec(memory_space=pltpu.MemorySpace.HBM),
            pl.BlockSpec(memory_space=pltpu.MemorySpace.HBM),
            pl.BlockSpec(memory_space=pltpu.MemorySpace.VMEM),
        ],
        out_specs=pl.BlockSpec(memory_space=pltpu.MemorySpace.HBM),
        out_shape=jax.ShapeDtypeStruct((n, d_out), jnp.float32),
        scratch_shapes=[
            pltpu.VMEM((n, d_in), jnp.float32),
            pltpu.VMEM((depth, m_blk, n), jnp.float32),
            pltpu.VMEM((o_slots, m_blk, d_out), jnp.float32),
            pltpu.SemaphoreType.DMA((depth,)),
            pltpu.SemaphoreType.DMA((o_slots,)),
            pltpu.SemaphoreType.DMA,
        ],
        compiler_params=pltpu.CompilerParams(
            vmem_limit_bytes=64 * 1024 * 1024),
        interpret=interpret,
    )(adj, x, W)


def kernel(x, adj, W):
    n = x.shape[0]
    if n % 80 == 0 and n // 80 >= 8:
        return _graph_conv(x, adj, W, m_blk=80, depth=6, o_slots=4)
    return _graph_conv(x, adj, W, m_blk=n, depth=1, o_slots=1)


# speedup vs baseline: 1.5703x; 1.0083x over previous
import functools

import jax
import jax.numpy as jnp
from jax.experimental import pallas as pl
from jax.experimental.pallas import tpu as pltpu


def _make_body(n, d_out, m_blk, depth, o_slots):
    n_blk = n // m_blk

    def _body(adj_hbm, x_hbm, w_ref, out_hbm,
              xbuf, buf, obuf, sems, osems, xsem):
        def cp_adj(i, slot):
            return pltpu.make_async_copy(
                adj_hbm.at[pl.ds(i * m_blk, m_blk), :],
                buf.at[slot],
                sems.at[slot],
            )

        def cp_out(i, oslot):
            return pltpu.make_async_copy(
                obuf.at[oslot],
                out_hbm.at[pl.ds(i * m_blk, m_blk), :],
                osems.at[oslot],
            )

        cp_x = pltpu.make_async_copy(x_hbm, xbuf, xsem)
        cp_x.start()

        for s in range(depth - 1):
            cp_adj(s, s).start()

        def step(i, carry):
            nxt = i + depth - 1

            @pl.when(nxt < n_blk)
            def _start_next():
                cp_adj(nxt, jax.lax.rem(nxt, depth)).start()

            slot = jax.lax.rem(i, depth)
            cp_adj(i, slot).wait()

            @pl.when(i == 0)
            def _wait_x():
                cp_x.wait()

            h = jnp.dot(buf[slot], xbuf[...],
                        preferred_element_type=jnp.float32)

            oslot = jax.lax.rem(i, o_slots)

            @pl.when(i >= o_slots)
            def _free_oslot():
                cp_out(i - o_slots, oslot).wait()

            obuf[oslot] = jax.lax.dot_general(
                h, w_ref[...], (((1,), (1,)), ((), ())),
                preferred_element_type=jnp.float32,
            )
            cp_out(i, oslot).start()
            return carry

        jax.lax.fori_loop(0, n_blk, step, 0)

        for tail in range(max(0, n_blk - o_slots), n_blk):
            cp_out(tail, tail % o_slots).wait()

    return _body


@functools.partial(jax.jit,
                   static_argnames=("m_blk", "depth", "o_slots", "interpret"))
def _graph_conv(x, adj, W, *, m_blk, depth, o_slots, interpret=False):
    n, d_in = x.shape
    d_out = W.shape[0]
    return pl.pallas_call(
        _make_body(n, d_out, m_blk, depth, o_slots),
        in_specs=[
            pl.BlockSpec(memory_space=pltpu.MemorySpace.HBM),
            pl.BlockSpec(memory_space=pltpu.MemorySpace.HBM),
            pl.BlockSpec(memory_space=pltpu.MemorySpace.VMEM),
        ],
        out_specs=pl.BlockSpec(memory_space=pltpu.MemorySpace.HBM),
        out_shape=jax.ShapeDtypeStruct((n, d_out), jnp.float32),
        scratch_shapes=[
            pltpu.VMEM((n, d_in), jnp.float32),
            pltpu.VMEM((depth, m_blk, n), jnp.float32),
            pltpu.VMEM((o_slots, m_blk, d_out), jnp.float32),
            pltpu.SemaphoreType.DMA((depth,)),
            pltpu.SemaphoreType.DMA((o_slots,)),
            pltpu.SemaphoreType.DMA,
        ],
        compiler_params=pltpu.CompilerParams(
            vmem_limit_bytes=64 * 1024 * 1024),
        interpret=interpret,
    )(adj, x, W)


def kernel(x, adj, W):
    n = x.shape[0]
    if n % 80 == 0 and n // 80 >= 8:
        return _graph_conv(x, adj, W, m_blk=80, depth=5, o_slots=4)
    return _graph_conv(x, adj, W, m_blk=n, depth=1, o_slots=1)
